# baseline (speedup 1.0000x reference)
import functools

import jax
import jax.numpy as jnp
from jax import lax
from jax.experimental import pallas as pl
from jax.experimental.pallas import tpu as pltpu
from jax.experimental.pallas import tpu_sc as plsc

_TM = 400


def _prop_body(a_ref, h_ref, w_ref, out_ref):
    y = jnp.dot(
        a_ref[...].astype(jnp.bfloat16),
        h_ref[...].astype(jnp.bfloat16),
        preferred_element_type=jnp.float32,
    )
    out_ref[...] = jnp.maximum(
        jnp.dot(y, w_ref[...], preferred_element_type=jnp.float32), 0.0
    )


def _propagate(A, H, W):
    n, d = H.shape
    return pl.pallas_call(
        _prop_body,
        grid=(n // _TM,),
        in_specs=[
            pl.BlockSpec((_TM, n), lambda i: (i, 0)),
            pl.BlockSpec((n, d), lambda i: (0, 0)),
            pl.BlockSpec((d, d), lambda i: (0, 0)),
        ],
        out_specs=pl.BlockSpec((_TM, d), lambda i: (i, 0)),
        out_shape=jax.ShapeDtypeStruct((n, d), jnp.float32),
    )(A, H, W)


_CH = 128
_L = 16


def _lane_shuffle(x, idx):
    dnums = lax.GatherDimensionNumbers(
        offset_dims=(), collapsed_slice_dims=(0,), start_index_map=(0,)
    )
    return lax.gather(
        x, idx[:, None], dnums, (1,),
        mode=lax.GatherScatterMode.PROMISE_IN_BOUNDS,
    )


def _pair_scores(H2, src_idx, dst_idx):
    b, d = src_idx.shape[0], H2.shape[1]
    info = plsc.get_sparse_core_info()
    nc, ns = info.num_cores, info.num_subcores
    nw = nc * ns
    per_w = b // nw
    nchunk = per_w // _CH
    mesh = plsc.VectorSubcoreMesh(core_axis_name="c", subcore_axis_name="s")

    @functools.partial(
        pl.kernel,
        mesh=mesh,
        out_type=jax.ShapeDtypeStruct((b,), jnp.float32),
        scratch_types=[
            pltpu.VMEM((_CH,), jnp.int32),
            pltpu.VMEM((_CH,), jnp.int32),
            pltpu.VMEM((_CH,), jnp.int32),
            pltpu.VMEM((_CH,), jnp.int32),
            pltpu.VMEM((_CH, d), jnp.float32),
            pltpu.VMEM((_CH, d), jnp.float32),
            pltpu.VMEM((_CH, d), jnp.float32),
            pltpu.VMEM((_CH, d), jnp.float32),
            pltpu.VMEM((_CH,), jnp.float32),
            pltpu.SemaphoreType.DMA,
            pltpu.SemaphoreType.DMA,
        ],
    )
    def body(h_hbm, src_hbm, dst_hbm, out_hbm,
             si0, si1, di0, di1, rs0, rs1, rd0, rd1, sc_v, s0, s1):
        src_bufs = (si0, si1)
        dst_bufs = (di0, di1)
        srow_bufs = (rs0, rs1)
        drow_bufs = (rd0, rd1)
        sems = (s0, s1)
        wid = lax.axis_index("s") * nc + lax.axis_index("c")
        base = wid * per_w

        def start(c):
            k = c % 2
            off = base + c * _CH
            pltpu.sync_copy(src_hbm.at[pl.ds(off, _CH)], src_bufs[k])
            pltpu.sync_copy(dst_hbm.at[pl.ds(off, _CH)], dst_bufs[k])
            pltpu.async_copy(h_hbm.at[src_bufs[k]], srow_bufs[k], sems[k])
            pltpu.async_copy(h_hbm.at[dst_bufs[k]], drow_bufs[k], sems[k])

        def finish(c):
            k = c % 2
            off = base + c * _CH
            pltpu.make_async_copy(h_hbm.at[src_bufs[k]], srow_bufs[k], sems[k]).wait()
            pltpu.make_async_copy(h_hbm.at[dst_bufs[k]], drow_bufs[k], sems[k]).wait()
            rs, rd = srow_bufs[k], drow_bufs[k]

            lane = lax.iota(jnp.int32, _L)

            def group(g, carry):
                vec = jnp.zeros((_L,), jnp.float32)
                for i in range(_L):
                    p = g * _L + i
                    acc = rs[p, pl.ds(0, _L)] * rd[p, pl.ds(0, _L)]
                    for j in range(1, d // _L):
                        acc = acc + rs[p, pl.ds(j * _L, _L)] * rd[p, pl.ds(j * _L, _L)]
                    for sh in (8, 4, 2, 1):
                        acc = acc + _lane_shuffle(acc, lane ^ sh)
                    vec = jnp.where(lane == i, acc, vec)
                sc_v[pl.ds(g * _L, _L)] = vec
                return carry

            lax.fori_loop(0, _CH // _L, group, 0, unroll=4)
            pltpu.sync_copy(sc_v, out_hbm.at[pl.ds(off, _CH)])

        start(0)
        for c in range(nchunk):
            if c + 1 < nchunk:
                start(c + 1)
            finish(c)

    return body(H2, src_idx, dst_idx)


def _loss_body(s_ref, lab_ref, out_ref):
    s = s_ref[...]
    lab = lab_ref[...]
    terms = lab * jax.nn.log_sigmoid(s) + (1.0 - lab) * jax.nn.log_sigmoid(-s)
    out_ref[...] = jnp.reshape(-jnp.sum(terms) / s.size, (1, 1))


def _loss(scores2d, labels2d):
    r, c = scores2d.shape
    return pl.pallas_call(
        _loss_body,
        in_specs=[
            pl.BlockSpec((r, c), lambda: (0, 0)),
            pl.BlockSpec((r, c), lambda: (0, 0)),
        ],
        out_specs=pl.BlockSpec((1, 1), lambda: (0, 0)),
        out_shape=jax.ShapeDtypeStruct((1, 1), jnp.float32),
    )(scores2d, labels2d)


def kernel(pairs, labels, A, embedding_state, W0, W1):
    H1 = _propagate(A, embedding_state, W0)
    H2 = _propagate(A, H1, W1)
    src_idx = pairs[:, 0].astype(jnp.int32)
    dst_idx = pairs[:, 1].astype(jnp.int32)
    scores = _pair_scores(H2, src_idx, dst_idx)
    loss2d = _loss(scores.reshape(128, -1), labels.reshape(128, -1))
    return loss2d[0, 0]


# speedup vs baseline: 1.0711x; 1.0711x over previous
import functools

import jax
import jax.numpy as jnp
from jax import lax
from jax.experimental import pallas as pl
from jax.experimental.pallas import tpu as pltpu
from jax.experimental.pallas import tpu_sc as plsc

_TM = 400


def _prop_body(a_ref, h_ref, w_ref, out_ref):
    y = jnp.dot(
        a_ref[...].astype(jnp.bfloat16),
        h_ref[...].astype(jnp.bfloat16),
        preferred_element_type=jnp.float32,
    )
    out_ref[...] = jnp.maximum(
        jnp.dot(y, w_ref[...], preferred_element_type=jnp.float32), 0.0
    )


def _propagate(A, H, W):
    n, d = H.shape
    return pl.pallas_call(
        _prop_body,
        grid=(n // _TM,),
        in_specs=[
            pl.BlockSpec((_TM, n), lambda i: (i, 0)),
            pl.BlockSpec((n, d), lambda i: (0, 0)),
            pl.BlockSpec((d, d), lambda i: (0, 0)),
        ],
        out_specs=pl.BlockSpec((_TM, d), lambda i: (i, 0)),
        out_shape=jax.ShapeDtypeStruct((n, d), jnp.float32),
    )(A, H, W)


_CH = 128
_L = 16


def _lane_shuffle(x, idx):
    dnums = lax.GatherDimensionNumbers(
        offset_dims=(), collapsed_slice_dims=(0,), start_index_map=(0,)
    )
    return lax.gather(
        x, idx[:, None], dnums, (1,),
        mode=lax.GatherScatterMode.PROMISE_IN_BOUNDS,
    )


def _pair_scores(H2, src_idx, dst_idx):
    b, d = src_idx.shape[0], H2.shape[1]
    info = plsc.get_sparse_core_info()
    nc, ns = info.num_cores, info.num_subcores
    nw = nc * ns
    per_w = b // nw
    nchunk = per_w // _CH
    mesh = plsc.VectorSubcoreMesh(core_axis_name="c", subcore_axis_name="s")

    @functools.partial(
        pl.kernel,
        mesh=mesh,
        out_type=jax.ShapeDtypeStruct((b,), jnp.float32),
        scratch_types=[
            pltpu.VMEM((_CH,), jnp.int32),
            pltpu.VMEM((_CH,), jnp.int32),
            pltpu.VMEM((_CH,), jnp.int32),
            pltpu.VMEM((_CH,), jnp.int32),
            pltpu.VMEM((_CH, d), jnp.float32),
            pltpu.VMEM((_CH, d), jnp.float32),
            pltpu.VMEM((_CH, d), jnp.float32),
            pltpu.VMEM((_CH, d), jnp.float32),
            pltpu.VMEM((_CH,), jnp.float32),
            pltpu.SemaphoreType.DMA,
            pltpu.SemaphoreType.DMA,
        ],
    )
    def body(h_hbm, src_hbm, dst_hbm, out_hbm,
             si0, si1, di0, di1, rs0, rs1, rd0, rd1, sc_v, s0, s1):
        src_bufs = (si0, si1)
        dst_bufs = (di0, di1)
        srow_bufs = (rs0, rs1)
        drow_bufs = (rd0, rd1)
        sems = (s0, s1)
        wid = lax.axis_index("s") * nc + lax.axis_index("c")
        base = wid * per_w

        def start(c):
            k = c % 2
            off = base + c * _CH
            pltpu.sync_copy(src_hbm.at[pl.ds(off, _CH)], src_bufs[k])
            pltpu.sync_copy(dst_hbm.at[pl.ds(off, _CH)], dst_bufs[k])
            pltpu.async_copy(h_hbm.at[src_bufs[k]], srow_bufs[k], sems[k])
            pltpu.async_copy(h_hbm.at[dst_bufs[k]], drow_bufs[k], sems[k])

        def finish(c):
            k = c % 2
            off = base + c * _CH
            pltpu.make_async_copy(h_hbm.at[src_bufs[k]], srow_bufs[k], sems[k]).wait()
            pltpu.make_async_copy(h_hbm.at[dst_bufs[k]], drow_bufs[k], sems[k]).wait()
            rs, rd = srow_bufs[k], drow_bufs[k]

            lane = lax.iota(jnp.int32, _L)

            def group(g, carry):
                vec = jnp.zeros((_L,), jnp.float32)
                for i in range(_L):
                    p = g * _L + i
                    acc = rs[p, pl.ds(0, _L)] * rd[p, pl.ds(0, _L)]
                    for j in range(1, d // _L):
                        acc = acc + rs[p, pl.ds(j * _L, _L)] * rd[p, pl.ds(j * _L, _L)]
                    for sh in (8, 4, 2, 1):
                        acc = acc + _lane_shuffle(acc, lane ^ sh)
                    vec = jnp.where(lane == i, acc, vec)
                sc_v[pl.ds(g * _L, _L)] = vec
                return carry

            lax.fori_loop(0, _CH // _L, group, 0, unroll=2)
            pltpu.sync_copy(sc_v, out_hbm.at[pl.ds(off, _CH)])

        start(0)
        for c in range(nchunk):
            if c + 1 < nchunk:
                start(c + 1)
            finish(c)

    return body(H2, src_idx, dst_idx)


def _loss_body(s_ref, lab_ref, out_ref):
    s = s_ref[...]
    lab = lab_ref[...]
    terms = lab * jax.nn.log_sigmoid(s) + (1.0 - lab) * jax.nn.log_sigmoid(-s)
    out_ref[...] = jnp.reshape(-jnp.sum(terms) / s.size, (1, 1))


def _loss(scores2d, labels2d):
    r, c = scores2d.shape
    return pl.pallas_call(
        _loss_body,
        in_specs=[
            pl.BlockSpec((r, c), lambda: (0, 0)),
            pl.BlockSpec((r, c), lambda: (0, 0)),
        ],
        out_specs=pl.BlockSpec((1, 1), lambda: (0, 0)),
        out_shape=jax.ShapeDtypeStruct((1, 1), jnp.float32),
    )(scores2d, labels2d)


def kernel(pairs, labels, A, embedding_state, W0, W1):
    H1 = _propagate(A, embedding_state, W0)
    H2 = _propagate(A, H1, W1)
    src_idx = pairs[:, 0].astype(jnp.int32)
    dst_idx = pairs[:, 1].astype(jnp.int32)
    scores = _pair_scores(H2, src_idx, dst_idx)
    loss2d = _loss(scores.reshape(128, -1), labels.reshape(128, -1))
    return loss2d[0, 0]
